# baseline (speedup 1.0000x reference)
import dataclasses
import functools

import jax
import jax.numpy as jnp
from jax import lax
from jax.experimental import pallas as pl
from jax.experimental.pallas import tpu as pltpu
from jax.experimental.pallas import tpu_sc as plsc

_L = 16
_NC = 2
_NS = 16
_NW = _NC * _NS
_BP = 2048
_TPB = _BP // 128
_SB = 64
_G = 8


def _labels(sd, st, cd, ct, vals):
    sa = st - sd
    ca = ct - cd
    cd1 = jnp.maximum(cd, 1.0)
    ca1 = jnp.maximum(ca, 1.0)
    known = (cd > 0.0) & (ca > 0.0)
    lo = jnp.where(known, jnp.where(sd * ca1 > sa * cd1, 0.0, 1.0), 2.0)
    black = lo == 0.0
    zero = jnp.zeros_like(vals)
    one = jnp.ones_like(vals)
    return lo, (jnp.where(black, vals, zero), jnp.where(black, one, zero),
                jnp.where(black, zero, vals), jnp.where(black, zero, one))


def _pass1_sc_body(img_v, edg_v, lo_v, val_v, acc_v):

    @pl.loop(0, _TPB)
    def _(pt):
        for k in range(8):
            psl = pl.ds(pt * 128 + k * 16, _L)
            lsl = pl.ds(k * _L, _L)
            zero = jnp.zeros((_L,), jnp.float32)
            sd = zero
            st = zero
            cd = zero
            ct = zero
            vals = zero
            for j in range(9):
                ev = edg_v[j, 0, pt, lsl]
                iv = img_v[j, 0, pt, lsl]
                dm = jnp.where(ev > 0.5, 1.0, 0.0)
                nz = jnp.where(ev != 0.0, 1.0, 0.0)
                sd = sd + iv * dm
                st = st + iv * nz
                cd = cd + dm
                ct = ct + nz
                if j == 4:
                    vals = iv
            lo, (pb, pcb, pw, pcw) = _labels(sd, st, cd, ct, vals)
            lo_v[psl] = lo
            val_v[psl] = vals
            plsc.addupdate(acc_v.at[0, pl.ds(0, _L)], pb)
            plsc.addupdate(acc_v.at[0, pl.ds(_L, _L)], pcb)
            plsc.addupdate(acc_v.at[0, pl.ds(2 * _L, _L)], pw)
            plsc.addupdate(acc_v.at[0, pl.ds(3 * _L, _L)], pcw)


def _sc_pass1(img_z, edg_z, n_sc):
    mesh = plsc.VectorSubcoreMesh(core_axis_name="c", subcore_axis_name="s")
    cp = pltpu.CompilerParams()
    if "needs_layout_passes" in pltpu.CompilerParams.__dataclass_fields__:
        cp = dataclasses.replace(cp, needs_layout_passes=False)

    @functools.partial(
        pl.kernel,
        compiler_params=cp,
        out_type=(
            jax.ShapeDtypeStruct((n_sc,), jnp.float32),
            jax.ShapeDtypeStruct((n_sc,), jnp.float32),
            jax.ShapeDtypeStruct((2 * _NW, 128), jnp.float32),
        ),
        mesh=mesh,
        scratch_types=[pltpu.VMEM((2, 128), jnp.float32)],
    )
    def k(img_hbm, edg_hbm, lo_hbm, val_hbm, part_hbm, acc_v):
        zero = jnp.zeros((_L,), jnp.float32)
        for q in range(16):
            acc_v[q // 8, pl.ds((q % 8) * _L, _L)] = zero

        def body(img_v, edg_v, lo_v, val_v):
            _pass1_sc_body(img_v, edg_v, lo_v, val_v, acc_v)

        blk = pl.BlockSpec((_BP,), lambda i: (i,))
        zblk = pl.BlockSpec((9, 1, _TPB, 128), lambda i: (0, i, 0, 0))
        pltpu.emit_pipeline(
            body,
            grid=(_SB,),
            in_specs=[zblk, zblk],
            out_specs=[blk, blk],
            core_axis_name=("c", "s"),
            dimension_semantics=(pltpu.PARALLEL,),
        )(img_hbm, edg_hbm, lo_hbm, val_hbm)

        wid = lax.axis_index("s") * _NC + lax.axis_index("c")
        pltpu.sync_copy(acc_v, part_hbm.at[pl.ds(2 * wid, 2)])

    return k(img_z, edg_z)


def _pass1_tc_kernel(img_ref, edg_ref, lo_ref, val_ref, part_ref, acc_s):
    i = pl.program_id(0)
    nsteps = pl.num_programs(0)

    @pl.when(i == 0)
    def _():
        for q in range(4):
            acc_s[q] = 0.0

    xi = img_ref[...]
    xe = edg_ref[...]
    dm = jnp.where(xe > 0.5, 1.0, 0.0)
    nz = jnp.where(xe != 0.0, 1.0, 0.0)
    sd = jnp.sum(xi * dm, axis=0)
    st = jnp.sum(xi * nz, axis=0)
    cd = jnp.sum(dm, axis=0)
    ct = jnp.sum(nz, axis=0)
    vals = xi[4]
    lo, (pb, pcb, pw, pcw) = _labels(sd, st, cd, ct, vals)
    lo_ref[...] = lo
    val_ref[...] = vals
    acc_s[0] += jnp.sum(pb)
    acc_s[1] += jnp.sum(pcb)
    acc_s[2] += jnp.sum(pw)
    acc_s[3] += jnp.sum(pcw)

    @pl.when(i == nsteps - 1)
    def _():
        lane = lax.broadcasted_iota(jnp.int32, (1, 4), 1)
        part_ref[...] = jnp.where(
            lane == 0, acc_s[0],
            jnp.where(lane == 1, acc_s[1],
                      jnp.where(lane == 2, acc_s[2], acc_s[3])))


def _tc_pass1(img_z, edg_z, n, n_sc):
    n_tc = n - n_sc
    nblocks = n_tc // (_BP * _G)
    zblk = pl.BlockSpec((9, _G, _TPB, 128), lambda i: (0, i, 0, 0))
    oblk = pl.BlockSpec((_G, _TPB, 128), lambda i: (i, 0, 0))
    return pl.pallas_call(
        _pass1_tc_kernel,
        grid=(nblocks,),
        in_specs=[zblk, zblk],
        out_specs=[oblk, oblk, pl.BlockSpec((1, 4), lambda i: (0, 0))],
        out_shape=(
            jax.ShapeDtypeStruct((nblocks * _G, _TPB, 128), jnp.float32),
            jax.ShapeDtypeStruct((nblocks * _G, _TPB, 128), jnp.float32),
            jax.ShapeDtypeStruct((1, 4), jnp.float32),
        ),
        scratch_shapes=[pltpu.SMEM((4,), jnp.float32)],
    )(img_z, edg_z)


_SC_BANDS = 4


def _pass2_kernel(lo_sc_ref, val_sc_ref, lo_tc_ref, val_tc_ref,
                  psc_ref, ptc_ref, out_ref):
    p = psc_ref[...]
    q = ptc_ref[...]
    sB = jnp.sum(p[:, 0 * _L:1 * _L]) + q[0, 0]
    cB = jnp.sum(p[:, 1 * _L:2 * _L]) + q[0, 1]
    sW = jnp.sum(p[:, 2 * _L:3 * _L]) + q[0, 2]
    cW = jnp.sum(p[:, 3 * _L:4 * _L]) + q[0, 3]
    avgB = sB / jnp.maximum(cB, 1.0)
    avgW = sW / jnp.maximum(cW, 1.0)
    i = pl.program_id(0)
    use_sc = i < _SC_BANDS
    lo = jnp.where(use_sc, lo_sc_ref[...], lo_tc_ref[...])
    v = jnp.where(use_sc, val_sc_ref[...], val_tc_ref[...])
    resolved = jnp.where(jnp.abs(v - avgB) < jnp.abs(v - avgW), 0.0, 1.0)
    corr = jnp.where(lo != 2.0, lo, resolved)
    out_ref[...] = corr.reshape(out_ref.shape)


def kernel(image, edges_prob, gt):
    global _SC_BANDS
    n = image.shape[0]
    H = gt.shape[0] - 2
    W = gt.shape[1] - 2
    nb = n // _BP

    n_sc = _SB * _BP

    def _z(x, lo_b, hi_b):
        return x[lo_b * _BP:hi_b * _BP].reshape(
            hi_b - lo_b, _TPB, 128, 9).transpose(3, 0, 1, 2)

    img_zs = _z(image, 0, _SB)
    edg_zs = _z(edges_prob, 0, _SB)
    img_zt = _z(image, _SB, nb)
    edg_zt = _z(edges_prob, _SB, nb)

    lo_sc, val_sc, parts_sc = _sc_pass1(img_zs, edg_zs, n_sc)
    lo_tc3, val_tc3, parts_tc = _tc_pass1(img_zt, edg_zt, n, n_sc)
    lo_tc = lo_tc3.reshape(-1)
    val_tc = val_tc3.reshape(-1)

    rows = 64
    band = rows * W
    _SC_BANDS = n_sc // band
    sc_bands = _SC_BANDS
    out = pl.pallas_call(
        _pass2_kernel,
        grid=(H // rows,),
        in_specs=[
            pl.BlockSpec((band,), lambda i: (jnp.minimum(i, sc_bands - 1),)),
            pl.BlockSpec((band,), lambda i: (jnp.minimum(i, sc_bands - 1),)),
            pl.BlockSpec(
                (band,), lambda i: (jnp.maximum(i - sc_bands, 0),)),
            pl.BlockSpec(
                (band,), lambda i: (jnp.maximum(i - sc_bands, 0),)),
            pl.BlockSpec((2 * _NW, 128), lambda i: (0, 0)),
            pl.BlockSpec((1, 4), lambda i: (0, 0)),
        ],
        out_specs=pl.BlockSpec((rows, W), lambda i: (i, 0)),
        out_shape=jax.ShapeDtypeStruct((H, W), jnp.float32),
    )(lo_sc, val_sc, lo_tc, val_tc, parts_sc, parts_tc)
    return out


# speedup vs baseline: 1.0682x; 1.0682x over previous
import dataclasses
import functools

import jax
import jax.numpy as jnp
from jax import lax
from jax.experimental import pallas as pl
from jax.experimental.pallas import tpu as pltpu
from jax.experimental.pallas import tpu_sc as plsc

_L = 16
_NC = 2
_NS = 16
_NW = _NC * _NS
_BP = 2048
_TPB = _BP // 128
_SB = 96
_G = 8


def _labels(sd, st, cd, ct, vals):
    sa = st - sd
    ca = ct - cd
    cd1 = jnp.maximum(cd, 1.0)
    ca1 = jnp.maximum(ca, 1.0)
    known = (cd > 0.0) & (ca > 0.0)
    lo = jnp.where(known, jnp.where(sd * ca1 > sa * cd1, 0.0, 1.0), 2.0)
    black = lo == 0.0
    zero = jnp.zeros_like(vals)
    one = jnp.ones_like(vals)
    return lo, (jnp.where(black, vals, zero), jnp.where(black, one, zero),
                jnp.where(black, zero, vals), jnp.where(black, zero, one))


def _pass1_sc_body(img_v, edg_v, lo_v, val_v, acc_v):

    @pl.loop(0, _TPB)
    def _(pt):
        for k in range(8):
            psl = pl.ds(pt * 128 + k * 16, _L)
            lsl = pl.ds(k * _L, _L)
            zero = jnp.zeros((_L,), jnp.float32)
            sd = zero
            st = zero
            cd = zero
            ct = zero
            vals = zero
            for j in range(9):
                ev = edg_v[j, 0, pt, lsl]
                iv = img_v[j, 0, pt, lsl]
                dm = jnp.where(ev > 0.5, 1.0, 0.0)
                nz = jnp.where(ev != 0.0, 1.0, 0.0)
                sd = sd + iv * dm
                st = st + iv * nz
                cd = cd + dm
                ct = ct + nz
                if j == 4:
                    vals = iv
            lo, (pb, pcb, pw, pcw) = _labels(sd, st, cd, ct, vals)
            lo_v[psl] = lo
            val_v[psl] = vals
            plsc.addupdate(acc_v.at[0, pl.ds(0, _L)], pb)
            plsc.addupdate(acc_v.at[0, pl.ds(_L, _L)], pcb)
            plsc.addupdate(acc_v.at[0, pl.ds(2 * _L, _L)], pw)
            plsc.addupdate(acc_v.at[0, pl.ds(3 * _L, _L)], pcw)


def _sc_pass1(img_z, edg_z, n_sc):
    mesh = plsc.VectorSubcoreMesh(core_axis_name="c", subcore_axis_name="s")
    cp = pltpu.CompilerParams()
    if "needs_layout_passes" in pltpu.CompilerParams.__dataclass_fields__:
        cp = dataclasses.replace(cp, needs_layout_passes=False)

    @functools.partial(
        pl.kernel,
        compiler_params=cp,
        out_type=(
            jax.ShapeDtypeStruct((n_sc,), jnp.float32),
            jax.ShapeDtypeStruct((n_sc,), jnp.float32),
            jax.ShapeDtypeStruct((2 * _NW, 128), jnp.float32),
        ),
        mesh=mesh,
        scratch_types=[pltpu.VMEM((2, 128), jnp.float32)],
    )
    def k(img_hbm, edg_hbm, lo_hbm, val_hbm, part_hbm, acc_v):
        zero = jnp.zeros((_L,), jnp.float32)
        for q in range(16):
            acc_v[q // 8, pl.ds((q % 8) * _L, _L)] = zero

        def body(img_v, edg_v, lo_v, val_v):
            _pass1_sc_body(img_v, edg_v, lo_v, val_v, acc_v)

        blk = pl.BlockSpec((_BP,), lambda i: (i,))
        zblk = pl.BlockSpec((9, 1, _TPB, 128), lambda i: (0, i, 0, 0))
        pltpu.emit_pipeline(
            body,
            grid=(_SB,),
            in_specs=[zblk, zblk],
            out_specs=[blk, blk],
            core_axis_name=("c", "s"),
            dimension_semantics=(pltpu.PARALLEL,),
        )(img_hbm, edg_hbm, lo_hbm, val_hbm)

        wid = lax.axis_index("s") * _NC + lax.axis_index("c")
        pltpu.sync_copy(acc_v, part_hbm.at[pl.ds(2 * wid, 2)])

    return k(img_z, edg_z)


def _pass1_tc_kernel(img_ref, edg_ref, lo_ref, val_ref, part_ref, acc_s):
    i = pl.program_id(0)
    nsteps = pl.num_programs(0)

    @pl.when(i == 0)
    def _():
        for q in range(4):
            acc_s[q] = 0.0

    xi = img_ref[...]
    xe = edg_ref[...]
    dm = jnp.where(xe > 0.5, 1.0, 0.0)
    nz = jnp.where(xe != 0.0, 1.0, 0.0)
    sd = jnp.sum(xi * dm, axis=0)
    st = jnp.sum(xi * nz, axis=0)
    cd = jnp.sum(dm, axis=0)
    ct = jnp.sum(nz, axis=0)
    vals = xi[4]
    lo, (pb, pcb, pw, pcw) = _labels(sd, st, cd, ct, vals)
    lo_ref[...] = lo
    val_ref[...] = vals
    acc_s[0] += jnp.sum(pb)
    acc_s[1] += jnp.sum(pcb)
    acc_s[2] += jnp.sum(pw)
    acc_s[3] += jnp.sum(pcw)

    @pl.when(i == nsteps - 1)
    def _():
        lane = lax.broadcasted_iota(jnp.int32, (1, 4), 1)
        part_ref[...] = jnp.where(
            lane == 0, acc_s[0],
            jnp.where(lane == 1, acc_s[1],
                      jnp.where(lane == 2, acc_s[2], acc_s[3])))


def _tc_pass1(img_z, edg_z, n, n_sc):
    n_tc = n - n_sc
    nblocks = n_tc // (_BP * _G)
    off = n_sc // (_BP * _G)
    zblk = pl.BlockSpec((9, _G, _TPB, 128), lambda i: (0, i + off, 0, 0))
    oblk = pl.BlockSpec((_G, _TPB, 128), lambda i: (i, 0, 0))
    return pl.pallas_call(
        _pass1_tc_kernel,
        grid=(nblocks,),
        in_specs=[zblk, zblk],
        out_specs=[oblk, oblk, pl.BlockSpec((1, 4), lambda i: (0, 0))],
        out_shape=(
            jax.ShapeDtypeStruct((nblocks * _G, _TPB, 128), jnp.float32),
            jax.ShapeDtypeStruct((nblocks * _G, _TPB, 128), jnp.float32),
            jax.ShapeDtypeStruct((1, 4), jnp.float32),
        ),
        scratch_shapes=[pltpu.SMEM((4,), jnp.float32)],
    )(img_z, edg_z)


_SC_BANDS = 4


def _pass2_kernel(lo_sc_ref, val_sc_ref, lo_tc_ref, val_tc_ref,
                  psc_ref, ptc_ref, out_ref):
    p = psc_ref[...]
    q = ptc_ref[...]
    sB = jnp.sum(p[:, 0 * _L:1 * _L]) + q[0, 0]
    cB = jnp.sum(p[:, 1 * _L:2 * _L]) + q[0, 1]
    sW = jnp.sum(p[:, 2 * _L:3 * _L]) + q[0, 2]
    cW = jnp.sum(p[:, 3 * _L:4 * _L]) + q[0, 3]
    avgB = sB / jnp.maximum(cB, 1.0)
    avgW = sW / jnp.maximum(cW, 1.0)
    i = pl.program_id(0)
    use_sc = i < _SC_BANDS
    lo = jnp.where(use_sc, lo_sc_ref[...], lo_tc_ref[...])
    v = jnp.where(use_sc, val_sc_ref[...], val_tc_ref[...])
    resolved = jnp.where(jnp.abs(v - avgB) < jnp.abs(v - avgW), 0.0, 1.0)
    corr = jnp.where(lo != 2.0, lo, resolved)
    out_ref[...] = corr.reshape(out_ref.shape)


def kernel(image, edges_prob, gt):
    global _SC_BANDS
    n = image.shape[0]
    H = gt.shape[0] - 2
    W = gt.shape[1] - 2
    nb = n // _BP

    img_z = image.reshape(nb, _TPB, 128, 9).transpose(3, 0, 1, 2)
    edg_z = edges_prob.reshape(nb, _TPB, 128, 9).transpose(3, 0, 1, 2)

    n_sc = _SB * _BP
    lo_sc, val_sc, parts_sc = _sc_pass1(img_z, edg_z, n_sc)
    lo_tc3, val_tc3, parts_tc = _tc_pass1(img_z, edg_z, n, n_sc)
    lo_tc = lo_tc3.reshape(-1)
    val_tc = val_tc3.reshape(-1)

    rows = 64
    band = rows * W
    _SC_BANDS = n_sc // band
    sc_bands = _SC_BANDS
    out = pl.pallas_call(
        _pass2_kernel,
        grid=(H // rows,),
        in_specs=[
            pl.BlockSpec((band,), lambda i: (jnp.minimum(i, sc_bands - 1),)),
            pl.BlockSpec((band,), lambda i: (jnp.minimum(i, sc_bands - 1),)),
            pl.BlockSpec(
                (band,), lambda i: (jnp.maximum(i - sc_bands, 0),)),
            pl.BlockSpec(
                (band,), lambda i: (jnp.maximum(i - sc_bands, 0),)),
            pl.BlockSpec((2 * _NW, 128), lambda i: (0, 0)),
            pl.BlockSpec((1, 4), lambda i: (0, 0)),
        ],
        out_specs=pl.BlockSpec((rows, W), lambda i: (i, 0)),
        out_shape=jax.ShapeDtypeStruct((H, W), jnp.float32),
    )(lo_sc, val_sc, lo_tc, val_tc, parts_sc, parts_tc)
    return out


# speedup vs baseline: 1.1558x; 1.0820x over previous
import dataclasses
import functools

import jax
import jax.numpy as jnp
from jax import lax
from jax.experimental import pallas as pl
from jax.experimental.pallas import tpu as pltpu
from jax.experimental.pallas import tpu_sc as plsc

_L = 16
_NC = 2
_NS = 16
_NW = _NC * _NS
_BP = 2048
_TPB = _BP // 128
_SB = 32
_G = 8


def _labels(sd, st, cd, ct, vals):
    sa = st - sd
    ca = ct - cd
    cd1 = jnp.maximum(cd, 1.0)
    ca1 = jnp.maximum(ca, 1.0)
    known = (cd > 0.0) & (ca > 0.0)
    lo = jnp.where(known, jnp.where(sd * ca1 > sa * cd1, 0.0, 1.0), 2.0)
    black = lo == 0.0
    zero = jnp.zeros_like(vals)
    one = jnp.ones_like(vals)
    return lo, (jnp.where(black, vals, zero), jnp.where(black, one, zero),
                jnp.where(black, zero, vals), jnp.where(black, zero, one))


def _pass1_sc_body(img_v, edg_v, lo_v, val_v, acc_v):

    @pl.loop(0, _TPB)
    def _(pt):
        for k in range(8):
            psl = pl.ds(pt * 128 + k * 16, _L)
            lsl = pl.ds(k * _L, _L)
            zero = jnp.zeros((_L,), jnp.float32)
            sd = zero
            st = zero
            cd = zero
            ct = zero
            vals = zero
            for j in range(9):
                ev = edg_v[j, 0, pt, lsl]
                iv = img_v[j, 0, pt, lsl]
                dm = jnp.where(ev > 0.5, 1.0, 0.0)
                nz = jnp.where(ev != 0.0, 1.0, 0.0)
                sd = sd + iv * dm
                st = st + iv * nz
                cd = cd + dm
                ct = ct + nz
                if j == 4:
                    vals = iv
            lo, (pb, pcb, pw, pcw) = _labels(sd, st, cd, ct, vals)
            lo_v[psl] = lo
            val_v[psl] = vals
            plsc.addupdate(acc_v.at[0, pl.ds(0, _L)], pb)
            plsc.addupdate(acc_v.at[0, pl.ds(_L, _L)], pcb)
            plsc.addupdate(acc_v.at[0, pl.ds(2 * _L, _L)], pw)
            plsc.addupdate(acc_v.at[0, pl.ds(3 * _L, _L)], pcw)


def _sc_pass1(img_z, edg_z, n_sc):
    mesh = plsc.VectorSubcoreMesh(core_axis_name="c", subcore_axis_name="s")
    cp = pltpu.CompilerParams()
    if "needs_layout_passes" in pltpu.CompilerParams.__dataclass_fields__:
        cp = dataclasses.replace(cp, needs_layout_passes=False)

    @functools.partial(
        pl.kernel,
        compiler_params=cp,
        out_type=(
            jax.ShapeDtypeStruct((n_sc,), jnp.float32),
            jax.ShapeDtypeStruct((n_sc,), jnp.float32),
            jax.ShapeDtypeStruct((2 * _NW, 128), jnp.float32),
        ),
        mesh=mesh,
        scratch_types=[pltpu.VMEM((2, 128), jnp.float32)],
    )
    def k(img_hbm, edg_hbm, lo_hbm, val_hbm, part_hbm, acc_v):
        zero = jnp.zeros((_L,), jnp.float32)
        for q in range(16):
            acc_v[q // 8, pl.ds((q % 8) * _L, _L)] = zero

        def body(img_v, edg_v, lo_v, val_v):
            _pass1_sc_body(img_v, edg_v, lo_v, val_v, acc_v)

        blk = pl.BlockSpec((_BP,), lambda i: (i,))
        zblk = pl.BlockSpec((9, 1, _TPB, 128), lambda i: (0, i, 0, 0))
        pltpu.emit_pipeline(
            body,
            grid=(_SB,),
            in_specs=[zblk, zblk],
            out_specs=[blk, blk],
            core_axis_name=("c", "s"),
            dimension_semantics=(pltpu.PARALLEL,),
        )(img_hbm, edg_hbm, lo_hbm, val_hbm)

        wid = lax.axis_index("s") * _NC + lax.axis_index("c")
        pltpu.sync_copy(acc_v, part_hbm.at[pl.ds(2 * wid, 2)])

    return k(img_z, edg_z)


def _pass1_tc_kernel(img_ref, edg_ref, lo_ref, val_ref, part_ref, acc_s):
    i = pl.program_id(0)
    nsteps = pl.num_programs(0)

    @pl.when(i == 0)
    def _():
        for q in range(4):
            acc_s[q] = 0.0

    xi = img_ref[...]
    xe = edg_ref[...]
    dm = jnp.where(xe > 0.5, 1.0, 0.0)
    nz = jnp.where(xe != 0.0, 1.0, 0.0)
    sd = jnp.sum(xi * dm, axis=0)
    st = jnp.sum(xi * nz, axis=0)
    cd = jnp.sum(dm, axis=0)
    ct = jnp.sum(nz, axis=0)
    vals = xi[4]
    lo, (pb, pcb, pw, pcw) = _labels(sd, st, cd, ct, vals)
    lo_ref[...] = lo
    val_ref[...] = vals
    acc_s[0] += jnp.sum(pb)
    acc_s[1] += jnp.sum(pcb)
    acc_s[2] += jnp.sum(pw)
    acc_s[3] += jnp.sum(pcw)

    @pl.when(i == nsteps - 1)
    def _():
        lane = lax.broadcasted_iota(jnp.int32, (1, 4), 1)
        part_ref[...] = jnp.where(
            lane == 0, acc_s[0],
            jnp.where(lane == 1, acc_s[1],
                      jnp.where(lane == 2, acc_s[2], acc_s[3])))


def _tc_pass1(img_z, edg_z, n, n_sc):
    n_tc = n - n_sc
    nblocks = n_tc // (_BP * _G)
    off = n_sc // (_BP * _G)
    zblk = pl.BlockSpec((9, _G, _TPB, 128), lambda i: (0, i + off, 0, 0))
    oblk = pl.BlockSpec((_G, _TPB, 128), lambda i: (i, 0, 0))
    return pl.pallas_call(
        _pass1_tc_kernel,
        grid=(nblocks,),
        in_specs=[zblk, zblk],
        out_specs=[oblk, oblk, pl.BlockSpec((1, 4), lambda i: (0, 0))],
        out_shape=(
            jax.ShapeDtypeStruct((nblocks * _G, _TPB, 128), jnp.float32),
            jax.ShapeDtypeStruct((nblocks * _G, _TPB, 128), jnp.float32),
            jax.ShapeDtypeStruct((1, 4), jnp.float32),
        ),
        scratch_shapes=[pltpu.SMEM((4,), jnp.float32)],
    )(img_z, edg_z)


_SC_BANDS = 4


def _pass2_kernel(lo_sc_ref, val_sc_ref, lo_tc_ref, val_tc_ref,
                  psc_ref, ptc_ref, out_ref):
    p = psc_ref[...]
    q = ptc_ref[...]
    sB = jnp.sum(p[:, 0 * _L:1 * _L]) + q[0, 0]
    cB = jnp.sum(p[:, 1 * _L:2 * _L]) + q[0, 1]
    sW = jnp.sum(p[:, 2 * _L:3 * _L]) + q[0, 2]
    cW = jnp.sum(p[:, 3 * _L:4 * _L]) + q[0, 3]
    avgB = sB / jnp.maximum(cB, 1.0)
    avgW = sW / jnp.maximum(cW, 1.0)
    i = pl.program_id(0)
    use_sc = i < _SC_BANDS
    lo = jnp.where(use_sc, lo_sc_ref[...], lo_tc_ref[...])
    v = jnp.where(use_sc, val_sc_ref[...], val_tc_ref[...])
    resolved = jnp.where(jnp.abs(v - avgB) < jnp.abs(v - avgW), 0.0, 1.0)
    corr = jnp.where(lo != 2.0, lo, resolved)
    out_ref[...] = corr.reshape(out_ref.shape)


def kernel(image, edges_prob, gt):
    global _SC_BANDS
    n = image.shape[0]
    H = gt.shape[0] - 2
    W = gt.shape[1] - 2
    nb = n // _BP

    img_z = image.reshape(nb, _TPB, 128, 9).transpose(3, 0, 1, 2)
    edg_z = edges_prob.reshape(nb, _TPB, 128, 9).transpose(3, 0, 1, 2)

    n_sc = _SB * _BP
    lo_sc, val_sc, parts_sc = _sc_pass1(img_z, edg_z, n_sc)
    lo_tc3, val_tc3, parts_tc = _tc_pass1(img_z, edg_z, n, n_sc)
    lo_tc = lo_tc3.reshape(-1)
    val_tc = val_tc3.reshape(-1)

    rows = 64
    band = rows * W
    _SC_BANDS = n_sc // band
    sc_bands = _SC_BANDS
    out = pl.pallas_call(
        _pass2_kernel,
        grid=(H // rows,),
        in_specs=[
            pl.BlockSpec((band,), lambda i: (jnp.minimum(i, sc_bands - 1),)),
            pl.BlockSpec((band,), lambda i: (jnp.minimum(i, sc_bands - 1),)),
            pl.BlockSpec(
                (band,), lambda i: (jnp.maximum(i - sc_bands, 0),)),
            pl.BlockSpec(
                (band,), lambda i: (jnp.maximum(i - sc_bands, 0),)),
            pl.BlockSpec((2 * _NW, 128), lambda i: (0, 0)),
            pl.BlockSpec((1, 4), lambda i: (0, 0)),
        ],
        out_specs=pl.BlockSpec((rows, W), lambda i: (i, 0)),
        out_shape=jax.ShapeDtypeStruct((H, W), jnp.float32),
    )(lo_sc, val_sc, lo_tc, val_tc, parts_sc, parts_tc)
    return out


# speedup vs baseline: 1.1838x; 1.0242x over previous
import dataclasses
import functools

import jax
import jax.numpy as jnp
from jax import lax
from jax.experimental import pallas as pl
from jax.experimental.pallas import tpu as pltpu
from jax.experimental.pallas import tpu_sc as plsc

_L = 16
_NC = 2
_NS = 16
_NW = _NC * _NS
_BP = 2048
_TPB = _BP // 128
_SB = 32
_G = 12


def _labels(sd, st, cd, ct, vals):
    sa = st - sd
    ca = ct - cd
    cd1 = jnp.maximum(cd, 1.0)
    ca1 = jnp.maximum(ca, 1.0)
    known = (cd > 0.0) & (ca > 0.0)
    lo = jnp.where(known, jnp.where(sd * ca1 > sa * cd1, 0.0, 1.0), 2.0)
    black = lo == 0.0
    zero = jnp.zeros_like(vals)
    one = jnp.ones_like(vals)
    return lo, (jnp.where(black, vals, zero), jnp.where(black, one, zero),
                jnp.where(black, zero, vals), jnp.where(black, zero, one))


def _pass1_sc_body(img_v, edg_v, lo_v, val_v, acc_v):

    @pl.loop(0, _TPB)
    def _(pt):
        for k in range(8):
            psl = pl.ds(pt * 128 + k * 16, _L)
            lsl = pl.ds(k * _L, _L)
            zero = jnp.zeros((_L,), jnp.float32)
            sd = zero
            st = zero
            cd = zero
            ct = zero
            vals = zero
            for j in range(9):
                ev = edg_v[j, 0, pt, lsl]
                iv = img_v[j, 0, pt, lsl]
                dm = jnp.where(ev > 0.5, 1.0, 0.0)
                nz = jnp.where(ev != 0.0, 1.0, 0.0)
                sd = sd + iv * dm
                st = st + iv * nz
                cd = cd + dm
                ct = ct + nz
                if j == 4:
                    vals = iv
            lo, (pb, pcb, pw, pcw) = _labels(sd, st, cd, ct, vals)
            lo_v[psl] = lo
            val_v[psl] = vals
            plsc.addupdate(acc_v.at[0, pl.ds(0, _L)], pb)
            plsc.addupdate(acc_v.at[0, pl.ds(_L, _L)], pcb)
            plsc.addupdate(acc_v.at[0, pl.ds(2 * _L, _L)], pw)
            plsc.addupdate(acc_v.at[0, pl.ds(3 * _L, _L)], pcw)


def _sc_pass1(img_z, edg_z, n_sc):
    mesh = plsc.VectorSubcoreMesh(core_axis_name="c", subcore_axis_name="s")
    cp = pltpu.CompilerParams()
    if "needs_layout_passes" in pltpu.CompilerParams.__dataclass_fields__:
        cp = dataclasses.replace(cp, needs_layout_passes=False)

    @functools.partial(
        pl.kernel,
        compiler_params=cp,
        out_type=(
            jax.ShapeDtypeStruct((n_sc,), jnp.float32),
            jax.ShapeDtypeStruct((n_sc,), jnp.float32),
            jax.ShapeDtypeStruct((2 * _NW, 128), jnp.float32),
        ),
        mesh=mesh,
        scratch_types=[pltpu.VMEM((2, 128), jnp.float32)],
    )
    def k(img_hbm, edg_hbm, lo_hbm, val_hbm, part_hbm, acc_v):
        zero = jnp.zeros((_L,), jnp.float32)
        for q in range(16):
            acc_v[q // 8, pl.ds((q % 8) * _L, _L)] = zero

        def body(img_v, edg_v, lo_v, val_v):
            _pass1_sc_body(img_v, edg_v, lo_v, val_v, acc_v)

        blk = pl.BlockSpec((_BP,), lambda i: (i,))
        zblk = pl.BlockSpec((9, 1, _TPB, 128), lambda i: (0, i, 0, 0))
        pltpu.emit_pipeline(
            body,
            grid=(_SB,),
            in_specs=[zblk, zblk],
            out_specs=[blk, blk],
            core_axis_name=("c", "s"),
            dimension_semantics=(pltpu.PARALLEL,),
        )(img_hbm, edg_hbm, lo_hbm, val_hbm)

        wid = lax.axis_index("s") * _NC + lax.axis_index("c")
        pltpu.sync_copy(acc_v, part_hbm.at[pl.ds(2 * wid, 2)])

    return k(img_z, edg_z)


def _pass1_tc_kernel(img_ref, edg_ref, lo_ref, val_ref, part_ref, acc_s):
    i = pl.program_id(0)
    nsteps = pl.num_programs(0)

    @pl.when(i == 0)
    def _():
        for q in range(4):
            acc_s[q] = 0.0

    xi = img_ref[...]
    xe = edg_ref[...]
    dm = jnp.where(xe > 0.5, 1.0, 0.0)
    nz = jnp.where(xe != 0.0, 1.0, 0.0)
    sd = jnp.sum(xi * dm, axis=0)
    st = jnp.sum(xi * nz, axis=0)
    cd = jnp.sum(dm, axis=0)
    ct = jnp.sum(nz, axis=0)
    vals = xi[4]
    lo, (pb, pcb, pw, pcw) = _labels(sd, st, cd, ct, vals)
    lo_ref[...] = lo
    val_ref[...] = vals
    acc_s[0] += jnp.sum(pb)
    acc_s[1] += jnp.sum(pcb)
    acc_s[2] += jnp.sum(pw)
    acc_s[3] += jnp.sum(pcw)

    @pl.when(i == nsteps - 1)
    def _():
        lane = lax.broadcasted_iota(jnp.int32, (1, 4), 1)
        part_ref[...] = jnp.where(
            lane == 0, acc_s[0],
            jnp.where(lane == 1, acc_s[1],
                      jnp.where(lane == 2, acc_s[2], acc_s[3])))


def _tc_pass1(img_z, edg_z, n, n_sc):
    n_tc = n - n_sc
    nblocks = n_tc // (_BP * _G)
    off = n_sc // (_BP * _G)
    zblk = pl.BlockSpec((9, _G, _TPB, 128), lambda i: (0, i + off, 0, 0))
    oblk = pl.BlockSpec((_G, _TPB, 128), lambda i: (i, 0, 0))
    return pl.pallas_call(
        _pass1_tc_kernel,
        grid=(nblocks,),
        in_specs=[zblk, zblk],
        out_specs=[oblk, oblk, pl.BlockSpec((1, 4), lambda i: (0, 0))],
        out_shape=(
            jax.ShapeDtypeStruct((nblocks * _G, _TPB, 128), jnp.float32),
            jax.ShapeDtypeStruct((nblocks * _G, _TPB, 128), jnp.float32),
            jax.ShapeDtypeStruct((1, 4), jnp.float32),
        ),
        scratch_shapes=[pltpu.SMEM((4,), jnp.float32)],
    )(img_z, edg_z)


_SC_BANDS = 4


def _pass2_kernel(lo_sc_ref, val_sc_ref, lo_tc_ref, val_tc_ref,
                  psc_ref, ptc_ref, out_ref):
    p = psc_ref[...]
    q = ptc_ref[...]
    sB = jnp.sum(p[:, 0 * _L:1 * _L]) + q[0, 0]
    cB = jnp.sum(p[:, 1 * _L:2 * _L]) + q[0, 1]
    sW = jnp.sum(p[:, 2 * _L:3 * _L]) + q[0, 2]
    cW = jnp.sum(p[:, 3 * _L:4 * _L]) + q[0, 3]
    avgB = sB / jnp.maximum(cB, 1.0)
    avgW = sW / jnp.maximum(cW, 1.0)
    i = pl.program_id(0)
    use_sc = i < _SC_BANDS
    lo = jnp.where(use_sc, lo_sc_ref[...], lo_tc_ref[...])
    v = jnp.where(use_sc, val_sc_ref[...], val_tc_ref[...])
    resolved = jnp.where(jnp.abs(v - avgB) < jnp.abs(v - avgW), 0.0, 1.0)
    corr = jnp.where(lo != 2.0, lo, resolved)
    out_ref[...] = corr.reshape(out_ref.shape)


def kernel(image, edges_prob, gt):
    global _SC_BANDS
    n = image.shape[0]
    H = gt.shape[0] - 2
    W = gt.shape[1] - 2
    nb = n // _BP

    img_z = image.reshape(nb, _TPB, 128, 9).transpose(3, 0, 1, 2)
    edg_z = edges_prob.reshape(nb, _TPB, 128, 9).transpose(3, 0, 1, 2)

    n_sc = _SB * _BP
    lo_sc, val_sc, parts_sc = _sc_pass1(img_z, edg_z, n_sc)
    lo_tc3, val_tc3, parts_tc = _tc_pass1(img_z, edg_z, n, n_sc)
    lo_tc = lo_tc3.reshape(-1)
    val_tc = val_tc3.reshape(-1)

    rows = 64
    band = rows * W
    _SC_BANDS = n_sc // band
    sc_bands = _SC_BANDS
    out = pl.pallas_call(
        _pass2_kernel,
        grid=(H // rows,),
        in_specs=[
            pl.BlockSpec((band,), lambda i: (jnp.minimum(i, sc_bands - 1),)),
            pl.BlockSpec((band,), lambda i: (jnp.minimum(i, sc_bands - 1),)),
            pl.BlockSpec(
                (band,), lambda i: (jnp.maximum(i - sc_bands, 0),)),
            pl.BlockSpec(
                (band,), lambda i: (jnp.maximum(i - sc_bands, 0),)),
            pl.BlockSpec((2 * _NW, 128), lambda i: (0, 0)),
            pl.BlockSpec((1, 4), lambda i: (0, 0)),
        ],
        out_specs=pl.BlockSpec((rows, W), lambda i: (i, 0)),
        out_shape=jax.ShapeDtypeStruct((H, W), jnp.float32),
    )(lo_sc, val_sc, lo_tc, val_tc, parts_sc, parts_tc)
    return out


# speedup vs baseline: 1.1963x; 1.0106x over previous
import dataclasses
import functools

import jax
import jax.numpy as jnp
from jax import lax
from jax.experimental import pallas as pl
from jax.experimental.pallas import tpu as pltpu
from jax.experimental.pallas import tpu_sc as plsc

_L = 16
_NC = 2
_NS = 16
_NW = _NC * _NS
_BP = 2048
_TPB = _BP // 128
_SB = 32
_G = 16


def _labels(sd, st, cd, ct, vals):
    sa = st - sd
    ca = ct - cd
    cd1 = jnp.maximum(cd, 1.0)
    ca1 = jnp.maximum(ca, 1.0)
    known = (cd > 0.0) & (ca > 0.0)
    lo = jnp.where(known, jnp.where(sd * ca1 > sa * cd1, 0.0, 1.0), 2.0)
    black = lo == 0.0
    zero = jnp.zeros_like(vals)
    one = jnp.ones_like(vals)
    return lo, (jnp.where(black, vals, zero), jnp.where(black, one, zero),
                jnp.where(black, zero, vals), jnp.where(black, zero, one))


def _pass1_sc_body(img_v, edg_v, lo_v, val_v, acc_v):

    @pl.loop(0, _TPB)
    def _(pt):
        for k in range(8):
            psl = pl.ds(pt * 128 + k * 16, _L)
            lsl = pl.ds(k * _L, _L)
            zero = jnp.zeros((_L,), jnp.float32)
            sd = zero
            st = zero
            cd = zero
            ct = zero
            vals = zero
            for j in range(9):
                ev = edg_v[j, 0, pt, lsl]
                iv = img_v[j, 0, pt, lsl]
                dm = jnp.where(ev > 0.5, 1.0, 0.0)
                nz = jnp.where(ev != 0.0, 1.0, 0.0)
                sd = sd + iv * dm
                st = st + iv * nz
                cd = cd + dm
                ct = ct + nz
                if j == 4:
                    vals = iv
            lo, (pb, pcb, pw, pcw) = _labels(sd, st, cd, ct, vals)
            lo_v[psl] = lo
            val_v[psl] = vals
            plsc.addupdate(acc_v.at[0, pl.ds(0, _L)], pb)
            plsc.addupdate(acc_v.at[0, pl.ds(_L, _L)], pcb)
            plsc.addupdate(acc_v.at[0, pl.ds(2 * _L, _L)], pw)
            plsc.addupdate(acc_v.at[0, pl.ds(3 * _L, _L)], pcw)


def _sc_pass1(img_z, edg_z, n_sc):
    mesh = plsc.VectorSubcoreMesh(core_axis_name="c", subcore_axis_name="s")
    cp = pltpu.CompilerParams()
    if "needs_layout_passes" in pltpu.CompilerParams.__dataclass_fields__:
        cp = dataclasses.replace(cp, needs_layout_passes=False)

    @functools.partial(
        pl.kernel,
        compiler_params=cp,
        out_type=(
            jax.ShapeDtypeStruct((n_sc,), jnp.float32),
            jax.ShapeDtypeStruct((n_sc,), jnp.float32),
            jax.ShapeDtypeStruct((2 * _NW, 128), jnp.float32),
        ),
        mesh=mesh,
        scratch_types=[pltpu.VMEM((2, 128), jnp.float32)],
    )
    def k(img_hbm, edg_hbm, lo_hbm, val_hbm, part_hbm, acc_v):
        zero = jnp.zeros((_L,), jnp.float32)
        for q in range(16):
            acc_v[q // 8, pl.ds((q % 8) * _L, _L)] = zero

        def body(img_v, edg_v, lo_v, val_v):
            _pass1_sc_body(img_v, edg_v, lo_v, val_v, acc_v)

        blk = pl.BlockSpec((_BP,), lambda i: (i,))
        zblk = pl.BlockSpec((9, 1, _TPB, 128), lambda i: (0, i, 0, 0))
        pltpu.emit_pipeline(
            body,
            grid=(_SB,),
            in_specs=[zblk, zblk],
            out_specs=[blk, blk],
            core_axis_name=("c", "s"),
            dimension_semantics=(pltpu.PARALLEL,),
        )(img_hbm, edg_hbm, lo_hbm, val_hbm)

        wid = lax.axis_index("s") * _NC + lax.axis_index("c")
        pltpu.sync_copy(acc_v, part_hbm.at[pl.ds(2 * wid, 2)])

    return k(img_z, edg_z)


def _pass1_tc_kernel(img_ref, edg_ref, lo_ref, val_ref, part_ref, acc_s):
    i = pl.program_id(0)
    nsteps = pl.num_programs(0)

    @pl.when(i == 0)
    def _():
        for q in range(4):
            acc_s[q] = 0.0

    xi = img_ref[...]
    xe = edg_ref[...]
    dm = jnp.where(xe > 0.5, 1.0, 0.0)
    nz = jnp.where(xe != 0.0, 1.0, 0.0)
    sd = jnp.sum(xi * dm, axis=0)
    st = jnp.sum(xi * nz, axis=0)
    cd = jnp.sum(dm, axis=0)
    ct = jnp.sum(nz, axis=0)
    vals = xi[4]
    lo, (pb, pcb, pw, pcw) = _labels(sd, st, cd, ct, vals)
    lo_ref[...] = lo
    val_ref[...] = vals
    acc_s[0] += jnp.sum(pb)
    acc_s[1] += jnp.sum(pcb)
    acc_s[2] += jnp.sum(pw)
    acc_s[3] += jnp.sum(pcw)

    @pl.when(i == nsteps - 1)
    def _():
        lane = lax.broadcasted_iota(jnp.int32, (1, 4), 1)
        part_ref[...] = jnp.where(
            lane == 0, acc_s[0],
            jnp.where(lane == 1, acc_s[1],
                      jnp.where(lane == 2, acc_s[2], acc_s[3])))


def _tc_pass1(img_z, edg_z, n, n_sc):
    n_tc = n - n_sc
    nblocks = n_tc // (_BP * _G)
    off = n_sc // (_BP * _G)
    zblk = pl.BlockSpec((9, _G, _TPB, 128), lambda i: (0, i + off, 0, 0))
    oblk = pl.BlockSpec((_G, _TPB, 128), lambda i: (i, 0, 0))
    return pl.pallas_call(
        _pass1_tc_kernel,
        grid=(nblocks,),
        in_specs=[zblk, zblk],
        out_specs=[oblk, oblk, pl.BlockSpec((1, 4), lambda i: (0, 0))],
        out_shape=(
            jax.ShapeDtypeStruct((nblocks * _G, _TPB, 128), jnp.float32),
            jax.ShapeDtypeStruct((nblocks * _G, _TPB, 128), jnp.float32),
            jax.ShapeDtypeStruct((1, 4), jnp.float32),
        ),
        scratch_shapes=[pltpu.SMEM((4,), jnp.float32)],
    )(img_z, edg_z)


_SC_BANDS = 4


def _pass2_kernel(lo_sc_ref, val_sc_ref, lo_tc_ref, val_tc_ref,
                  psc_ref, ptc_ref, out_ref):
    p = psc_ref[...]
    q = ptc_ref[...]
    sB = jnp.sum(p[:, 0 * _L:1 * _L]) + q[0, 0]
    cB = jnp.sum(p[:, 1 * _L:2 * _L]) + q[0, 1]
    sW = jnp.sum(p[:, 2 * _L:3 * _L]) + q[0, 2]
    cW = jnp.sum(p[:, 3 * _L:4 * _L]) + q[0, 3]
    avgB = sB / jnp.maximum(cB, 1.0)
    avgW = sW / jnp.maximum(cW, 1.0)
    i = pl.program_id(0)
    use_sc = i < _SC_BANDS
    lo = jnp.where(use_sc, lo_sc_ref[...], lo_tc_ref[...])
    v = jnp.where(use_sc, val_sc_ref[...], val_tc_ref[...])
    resolved = jnp.where(jnp.abs(v - avgB) < jnp.abs(v - avgW), 0.0, 1.0)
    corr = jnp.where(lo != 2.0, lo, resolved)
    out_ref[...] = corr.reshape(out_ref.shape)


def kernel(image, edges_prob, gt):
    global _SC_BANDS
    n = image.shape[0]
    H = gt.shape[0] - 2
    W = gt.shape[1] - 2
    nb = n // _BP

    img_z = image.reshape(nb, _TPB, 128, 9).transpose(3, 0, 1, 2)
    edg_z = edges_prob.reshape(nb, _TPB, 128, 9).transpose(3, 0, 1, 2)

    n_sc = _SB * _BP
    lo_sc, val_sc, parts_sc = _sc_pass1(img_z, edg_z, n_sc)
    lo_tc3, val_tc3, parts_tc = _tc_pass1(img_z, edg_z, n, n_sc)
    lo_tc = lo_tc3.reshape(-1)
    val_tc = val_tc3.reshape(-1)

    rows = 64
    band = rows * W
    _SC_BANDS = n_sc // band
    sc_bands = _SC_BANDS
    out = pl.pallas_call(
        _pass2_kernel,
        grid=(H // rows,),
        in_specs=[
            pl.BlockSpec((band,), lambda i: (jnp.minimum(i, sc_bands - 1),)),
            pl.BlockSpec((band,), lambda i: (jnp.minimum(i, sc_bands - 1),)),
            pl.BlockSpec(
                (band,), lambda i: (jnp.maximum(i - sc_bands, 0),)),
            pl.BlockSpec(
                (band,), lambda i: (jnp.maximum(i - sc_bands, 0),)),
            pl.BlockSpec((2 * _NW, 128), lambda i: (0, 0)),
            pl.BlockSpec((1, 4), lambda i: (0, 0)),
        ],
        out_specs=pl.BlockSpec((rows, W), lambda i: (i, 0)),
        out_shape=jax.ShapeDtypeStruct((H, W), jnp.float32),
    )(lo_sc, val_sc, lo_tc, val_tc, parts_sc, parts_tc)
    return out
